# baseline (device time: 18817 ns/iter reference)
import jax
import jax.numpy as jnp
from jax import lax
from jax.experimental import pallas as pl
from jax.experimental.pallas import tpu as pltpu

N_DEV = 4
N_LOCAL_EXPERTS = 4
N_TOKENS = 1024
D_MODEL = 256
D_HID = 512
CHUNK = N_TOKENS // N_DEV


def kernel(x, router_W, route_idx, expert_W, shared_W):
    def body(x_ref, router_ref, idx_ref, ew_ref, sw_ref, out_ref,
             send_buf, comm_ref, send_sems, recv_sems):
        my = lax.axis_index("i")

        barrier_sem = pltpu.get_barrier_semaphore()
        for k in range(1, N_DEV):
            pl.semaphore_signal(
                barrier_sem, inc=1,
                device_id=((my + k) % N_DEV,),
                device_id_type=pl.DeviceIdType.MESH,
            )
        pl.semaphore_wait(barrier_sem, N_DEV - 1)

        ew_flat = ew_ref[:, :, :].reshape(N_LOCAL_EXPERTS * D_MODEL, D_HID)

        def chunk_contrib(t):
            xc = x_ref[pl.ds(t * CHUNK, CHUNK), :]
            idx = idx_ref[pl.ds(t * CHUNK, CHUNK), :]
            scores = jnp.dot(xc, router_ref[:, :],
                             preferred_element_type=jnp.float32)
            s_max = jnp.max(scores, axis=-1, keepdims=True)
            e_s = jnp.exp(scores - s_max)
            probs = e_s / jnp.sum(e_s, axis=-1, keepdims=True)
            col = lax.broadcasted_iota(jnp.int32, scores.shape, 1)
            gate = jnp.sum(jnp.where(col == idx, probs, 0.0), axis=-1,
                           keepdims=True)
            xm = jnp.concatenate(
                [jnp.where(idx == my * N_LOCAL_EXPERTS + e, gate, 0.0) * xc
                 for e in range(N_LOCAL_EXPERTS)],
                axis=1)
            return jnp.dot(xm, ew_flat,
                           preferred_element_type=jnp.float32)

        rdmas = []
        for k in (2, 1, 3):
            t = (my + k) % N_DEV
            send_buf[k - 1, :, :] = chunk_contrib(t).astype(jnp.bfloat16)
            rdma = pltpu.make_async_remote_copy(
                src_ref=send_buf.at[k - 1],
                dst_ref=comm_ref.at[3 - k],
                send_sem=send_sems.at[k - 1],
                recv_sem=recv_sems.at[3 - k],
                device_id=(t,),
                device_id_type=pl.DeviceIdType.MESH,
            )
            rdma.start()
            rdmas.append(rdma)

        own = chunk_contrib(my)
        x_own = x_ref[pl.ds(my * CHUNK, CHUNK), :]
        shared = jnp.dot(x_own, sw_ref[:, :],
                         preferred_element_type=jnp.float32)

        for j in range(N_DEV - 1):
            recv = pltpu.make_async_remote_copy(
                src_ref=send_buf.at[0],
                dst_ref=comm_ref.at[j],
                send_sem=send_sems.at[0],
                recv_sem=recv_sems.at[j],
                device_id=(my,),
                device_id_type=pl.DeviceIdType.MESH,
            )
            recv.wait_recv()

        out_ref[:, :] = (
            own + shared
            + comm_ref[0, :, :].astype(jnp.float32)
            + comm_ref[1, :, :].astype(jnp.float32)
            + comm_ref[2, :, :].astype(jnp.float32)
        )

        for rdma in rdmas:
            rdma.wait_send()

    return pl.pallas_call(
        body,
        out_shape=jax.ShapeDtypeStruct((CHUNK, D_HID), jnp.float32),
        in_specs=[pl.BlockSpec(memory_space=pltpu.VMEM)] * 5,
        out_specs=pl.BlockSpec(memory_space=pltpu.VMEM),
        scratch_shapes=[
            pltpu.VMEM((N_DEV - 1, CHUNK, D_HID), jnp.bfloat16),
            pltpu.VMEM((N_DEV - 1, CHUNK, D_HID), jnp.bfloat16),
            pltpu.SemaphoreType.DMA((N_DEV - 1,)),
            pltpu.SemaphoreType.DMA((N_DEV - 1,)),
        ],
        compiler_params=pltpu.CompilerParams(collective_id=0),
    )(x, router_W, route_idx, expert_W, shared_W)
